# baseline (speedup 1.0000x reference)
import functools

import jax
import jax.numpy as jnp
from jax import lax
from jax.experimental import pallas as pl
from jax.experimental.pallas import tpu as pltpu
from jax.experimental.pallas import tpu_sc as plsc

N = 10000
E = 320000
F_IN = 128
F_HID = 64
N_CLASS = 16

NP = 10240
PAD_ROW = N
CHUNK = 128
EPAD = 327680
CPT = EPAD // (32 * CHUNK)
RPT = NP // 16

_MESH = plsc.VectorSubcoreMesh(core_axis_name="c", subcore_axis_name="s")
_SC_PARAMS = pltpu.CompilerParams(use_tc_tiling_on_sc=False)


@functools.partial(
    pl.kernel,
    out_type=jax.ShapeDtypeStruct((2, NP), jnp.float32),
    mesh=_MESH,
    compiler_params=_SC_PARAMS,
    scratch_types=[
        pltpu.VMEM((CPT, CHUNK), jnp.int32),
        pltpu.VMEM((CHUNK,), jnp.float32),
        pltpu.VMEM_SHARED((NP,), jnp.float32),
    ],
)
def _sc_degree(dst_hbm, z_hbm, out_hbm, dst_v, ones_v, acc):
    c = lax.axis_index("c")
    s = lax.axis_index("s")
    wid = c * 16 + s
    pltpu.sync_copy(z_hbm.at[pl.ds(s * RPT, RPT)], acc.at[pl.ds(s * RPT, RPT)])
    for i in range(CHUNK // 16):
        ones_v[pl.ds(i * 16, 16)] = jnp.full((16,), 1.0, jnp.float32)
    pltpu.sync_copy(dst_hbm.at[pl.ds(wid * CPT, CPT)], dst_v)
    plsc.subcore_barrier()

    def body(j, carry):
        pltpu.sync_copy(ones_v, acc.at[dst_v.at[j]], add=True)
        return carry

    lax.fori_loop(0, CPT, body, 0)
    plsc.subcore_barrier()
    pltpu.sync_copy(acc.at[pl.ds(s * RPT, RPT)], out_hbm.at[c, pl.ds(s * RPT, RPT)])


NBUF = 5
CPT0 = 160
CPT1 = 160 - CPT0


def _make_sc_spmm(width):
    @functools.partial(
        pl.kernel,
        out_type=jax.ShapeDtypeStruct((2, NP, width), jnp.float32),
        mesh=_MESH,
        compiler_params=_SC_PARAMS,
        scratch_types=[
            pltpu.VMEM((max(CPT0, CPT1), CHUNK), jnp.int32),
            pltpu.VMEM((max(CPT0, CPT1), CHUNK), jnp.int32),
            pltpu.VMEM((NBUF, CHUNK, width), jnp.float32),
            pltpu.VMEM_SHARED((NP, width), jnp.float32),
        ] + [pltpu.SemaphoreType.DMA] * (2 * NBUF),
    )
    def spmm(s_hbm, src_hbm, dst_hbm, z_hbm, out_hbm,
             src_v, dst_v, rows_v, acc, *sems):
        gsems = sems[:NBUF]
        ssems = sems[NBUF:]
        c = lax.axis_index("c")
        s = lax.axis_index("s")
        pltpu.sync_copy(z_hbm.at[pl.ds(s * RPT, RPT)], acc.at[pl.ds(s * RPT, RPT)])
        plsc.subcore_barrier()

        def run(base, count):
            pltpu.sync_copy(src_hbm.at[pl.ds(base, count)],
                            src_v.at[pl.ds(0, count)])
            pltpu.sync_copy(dst_hbm.at[pl.ds(base, count)],
                            dst_v.at[pl.ds(0, count)])
            for b in range(NBUF):
                pltpu.async_copy(s_hbm.at[src_v.at[b]], rows_v.at[b], gsems[b])

            def body(g, carry):
                gb = g * NBUF
                for b in range(NBUF):
                    j = gb + b
                    pltpu.make_async_copy(
                        s_hbm.at[src_v.at[j]], rows_v.at[b], gsems[b]).wait()
                    pltpu.async_copy(
                        rows_v.at[b], acc.at[dst_v.at[j]], ssems[b], add=True)
                for b in range(NBUF):
                    j = gb + b
                    pltpu.make_async_copy(
                        rows_v.at[b], acc.at[dst_v.at[j]], ssems[b]).wait()
                    nj = j + NBUF

                    @pl.when(nj < count)
                    def _(b=b, nj=nj):
                        pltpu.async_copy(
                            s_hbm.at[src_v.at[nj]], rows_v.at[b], gsems[b])
                return carry

            lax.fori_loop(0, count // NBUF, body, 0)

        @pl.when(c == 0)
        def _():
            run(s * CPT0, CPT0)

        if CPT1:
            @pl.when(c == 1)
            def _():
                run(16 * CPT0 + s * CPT1, CPT1)
        plsc.subcore_barrier()
        pltpu.sync_copy(acc.at[pl.ds(s * RPT, RPT)],
                        out_hbm.at[c, pl.ds(s * RPT, RPT)])

    return spmm


_sc_spmm64 = _make_sc_spmm(F_HID)
_sc_spmm16 = _make_sc_spmm(N_CLASS)


_RB = 2048


def _dinv(d_ref):
    return lax.rsqrt(d_ref[0, :] + d_ref[1, :] + 1.0)


def _tc1_body(x_ref, w_ref, d_ref, o_ref):
    s = jnp.dot(x_ref[...], w_ref[...], preferred_element_type=jnp.float32)
    o_ref[...] = s * _dinv(d_ref)[:, None]


def _tc2_body(h_ref, d_ref, w_ref, o_ref):
    dinv = _dinv(d_ref)
    h1 = (h_ref[0] + h_ref[1]) * dinv[:, None]
    s = jnp.dot(h1, w_ref[...], preferred_element_type=jnp.float32)
    o_ref[...] = s * dinv[:, None]


def _tc3_body(h_ref, d_ref, o_ref):
    h2 = (h_ref[0] + h_ref[1]) * _dinv(d_ref)[:, None]
    m = jnp.max(h2, axis=1, keepdims=True)
    ex = jnp.exp(h2 - m)
    lse = jnp.log(jnp.sum(ex, axis=1, keepdims=True)) + m
    o_ref[...] = h2 - lse


def _tc1(xp, W1, degp):
    return pl.pallas_call(
        _tc1_body,
        grid=(NP // _RB,),
        in_specs=[
            pl.BlockSpec((_RB, F_IN), lambda i: (i, 0)),
            pl.BlockSpec((F_IN, F_HID), lambda i: (0, 0)),
            pl.BlockSpec((2, _RB), lambda i: (0, i)),
        ],
        out_specs=pl.BlockSpec((_RB, F_HID), lambda i: (i, 0)),
        out_shape=jax.ShapeDtypeStruct((NP, F_HID), jnp.float32),
    )(xp, W1, degp)


def _tc2(h1p, degp, W2):
    return pl.pallas_call(
        _tc2_body,
        grid=(NP // _RB,),
        in_specs=[
            pl.BlockSpec((2, _RB, F_HID), lambda i: (0, i, 0)),
            pl.BlockSpec((2, _RB), lambda i: (0, i)),
            pl.BlockSpec((F_HID, N_CLASS), lambda i: (0, 0)),
        ],
        out_specs=pl.BlockSpec((_RB, N_CLASS), lambda i: (i, 0)),
        out_shape=jax.ShapeDtypeStruct((NP, N_CLASS), jnp.float32),
    )(h1p, degp, W2)


def _tc3(h2p, degp):
    return pl.pallas_call(
        _tc3_body,
        grid=(NP // _RB,),
        in_specs=[
            pl.BlockSpec((2, _RB, N_CLASS), lambda i: (0, i, 0)),
            pl.BlockSpec((2, _RB), lambda i: (0, i)),
        ],
        out_specs=pl.BlockSpec((_RB, N_CLASS), lambda i: (i, 0)),
        out_shape=jax.ShapeDtypeStruct((NP, N_CLASS), jnp.float32),
    )(h2p, degp)


@jax.jit
def kernel(x, edge_index, W1, W2):
    pad = jnp.full((EPAD - E,), PAD_ROW, jnp.int32)
    src2d = jnp.concatenate([edge_index[0], pad]).reshape(EPAD // CHUNK, CHUNK)
    dst2d = jnp.concatenate([edge_index[1], pad]).reshape(EPAD // CHUNK, CHUNK)
    xp = jnp.pad(x, ((0, NP - N), (0, 0)))
    z1 = jnp.zeros((NP,), jnp.float32)
    z64 = jnp.zeros((NP, F_HID), jnp.float32)
    z16 = jnp.zeros((NP, N_CLASS), jnp.float32)

    degp = _sc_degree(dst2d, z1)
    s1 = _tc1(xp, W1, degp)
    h1p = _sc_spmm64(s1, src2d, dst2d, z64)
    s2 = _tc2(h1p, degp, W2)
    h2p = _sc_spmm16(s2, src2d, dst2d, z16)
    out = _tc3(h2p, degp)
    return out[:N]


# speedup vs baseline: 1.1588x; 1.1588x over previous
import functools

import jax
import jax.numpy as jnp
from jax import lax
from jax.experimental import pallas as pl
from jax.experimental.pallas import tpu as pltpu
from jax.experimental.pallas import tpu_sc as plsc

N = 10000
E = 320000
F_IN = 128
F_HID = 64
N_CLASS = 16

NP = 10240
PAD_ROW = N
CHUNK = 128
EPAD = 327680
CPT = EPAD // (32 * CHUNK)
RPT = NP // 16

_MESH = plsc.VectorSubcoreMesh(core_axis_name="c", subcore_axis_name="s")
_SC_PARAMS = pltpu.CompilerParams(use_tc_tiling_on_sc=False)


@functools.partial(
    pl.kernel,
    out_type=jax.ShapeDtypeStruct((2, NP), jnp.float32),
    mesh=_MESH,
    compiler_params=_SC_PARAMS,
    scratch_types=[
        pltpu.VMEM((CPT, CHUNK), jnp.int32),
        pltpu.VMEM((CHUNK,), jnp.float32),
        pltpu.VMEM_SHARED((NP,), jnp.float32),
    ],
)
def _sc_degree(dst_hbm, z_hbm, out_hbm, dst_v, ones_v, acc):
    c = lax.axis_index("c")
    s = lax.axis_index("s")
    wid = c * 16 + s
    pltpu.sync_copy(z_hbm.at[pl.ds(s * RPT, RPT)], acc.at[pl.ds(s * RPT, RPT)])
    for i in range(CHUNK // 16):
        ones_v[pl.ds(i * 16, 16)] = jnp.full((16,), 1.0, jnp.float32)
    pltpu.sync_copy(dst_hbm.at[pl.ds(wid * CPT, CPT)], dst_v)
    plsc.subcore_barrier()

    def body(j, carry):
        pltpu.sync_copy(ones_v, acc.at[dst_v.at[j]], add=True)
        return carry

    lax.fori_loop(0, CPT, body, 0)
    plsc.subcore_barrier()
    pltpu.sync_copy(acc.at[pl.ds(s * RPT, RPT)], out_hbm.at[c, pl.ds(s * RPT, RPT)])


NBUF = 8
CPT0 = 96
CPT1 = 160 - CPT0


def _make_sc_spmm(width):
    @functools.partial(
        pl.kernel,
        out_type=jax.ShapeDtypeStruct((2, NP, width), jnp.float32),
        mesh=_MESH,
        compiler_params=_SC_PARAMS,
        scratch_types=[
            pltpu.VMEM((2, NBUF, CHUNK), jnp.int32),
            pltpu.VMEM((2, NBUF, CHUNK), jnp.int32),
            pltpu.VMEM((NBUF, CHUNK, width), jnp.float32),
            pltpu.VMEM_SHARED((NP, width), jnp.float32),
        ] + [pltpu.SemaphoreType.DMA] * (2 * NBUF + 2),
    )
    def spmm(s_hbm, src_hbm, dst_hbm, z_hbm, out_hbm,
             src_v, dst_v, rows_v, acc, *sems):
        gsems = sems[:NBUF]
        ssems = sems[NBUF:2 * NBUF]
        isems = sems[2 * NBUF:]
        c = lax.axis_index("c")
        s = lax.axis_index("s")
        pltpu.sync_copy(z_hbm.at[pl.ds(s * RPT, RPT)], acc.at[pl.ds(s * RPT, RPT)])
        plsc.subcore_barrier()

        def idx_load(slot, blk):
            pltpu.async_copy(src_hbm.at[pl.ds(blk * NBUF, NBUF)],
                             src_v.at[slot], isems[slot])
            pltpu.async_copy(dst_hbm.at[pl.ds(blk * NBUF, NBUF)],
                             dst_v.at[slot], isems[slot])

        def idx_wait(slot, blk):
            pltpu.make_async_copy(src_hbm.at[pl.ds(blk * NBUF, NBUF)],
                                  src_v.at[slot], isems[slot]).wait()
            pltpu.make_async_copy(dst_hbm.at[pl.ds(blk * NBUF, NBUF)],
                                  dst_v.at[slot], isems[slot]).wait()

        def run(base_blk, count):
            G = count // NBUF
            idx_load(0, base_blk)
            idx_load(1, base_blk + 1)
            idx_wait(0, base_blk)
            for b in range(NBUF):
                pltpu.async_copy(s_hbm.at[src_v.at[0, b]], rows_v.at[b],
                                 gsems[b])

            def group(g, slot):
                nslot = 1 - slot
                for b in range(NBUF):
                    pltpu.make_async_copy(
                        s_hbm.at[src_v.at[slot, b]], rows_v.at[b],
                        gsems[b]).wait()
                    pltpu.async_copy(
                        rows_v.at[b], acc.at[dst_v.at[slot, b]], ssems[b],
                        add=True)
                @pl.when(g + 1 < G)
                def _():
                    idx_wait(nslot, base_blk + g + 1)
                for b in range(NBUF):
                    pltpu.make_async_copy(
                        rows_v.at[b], acc.at[dst_v.at[slot, b]],
                        ssems[b]).wait()

                    @pl.when(g + 1 < G)
                    def _(b=b):
                        pltpu.async_copy(
                            s_hbm.at[src_v.at[nslot, b]], rows_v.at[b],
                            gsems[b])
                @pl.when(g + 2 < G)
                def _():
                    idx_load(slot, base_blk + g + 2)

            def body(h, carry):
                group(2 * h, 0)
                group(2 * h + 1, 1)
                return carry

            lax.fori_loop(0, G // 2, body, 0)

        @pl.when(c == 0)
        def _():
            run(s * (CPT0 // NBUF), CPT0)

        if CPT1:
            @pl.when(c == 1)
            def _():
                run((16 * CPT0 + s * CPT1) // NBUF, CPT1)
        plsc.subcore_barrier()
        pltpu.sync_copy(acc.at[pl.ds(s * RPT, RPT)],
                        out_hbm.at[c, pl.ds(s * RPT, RPT)])

    return spmm


_sc_spmm64 = _make_sc_spmm(F_HID)
_sc_spmm16 = _make_sc_spmm(N_CLASS)


_RB = 2048


def _dinv(d_ref):
    return lax.rsqrt(d_ref[0, :] + d_ref[1, :] + 1.0)


def _tc1_body(x_ref, w_ref, d_ref, o_ref):
    s = jnp.dot(x_ref[...], w_ref[...], preferred_element_type=jnp.float32)
    o_ref[...] = s * _dinv(d_ref)[:, None]


def _tc2_body(h_ref, d_ref, w_ref, o_ref):
    dinv = _dinv(d_ref)
    h1 = (h_ref[0] + h_ref[1]) * dinv[:, None]
    s = jnp.dot(h1, w_ref[...], preferred_element_type=jnp.float32)
    o_ref[...] = s * dinv[:, None]


def _tc3_body(h_ref, d_ref, o_ref):
    h2 = (h_ref[0] + h_ref[1]) * _dinv(d_ref)[:, None]
    m = jnp.max(h2, axis=1, keepdims=True)
    ex = jnp.exp(h2 - m)
    lse = jnp.log(jnp.sum(ex, axis=1, keepdims=True)) + m
    o_ref[...] = h2 - lse


def _tc1(xp, W1, degp):
    return pl.pallas_call(
        _tc1_body,
        grid=(NP // _RB,),
        in_specs=[
            pl.BlockSpec((_RB, F_IN), lambda i: (i, 0)),
            pl.BlockSpec((F_IN, F_HID), lambda i: (0, 0)),
            pl.BlockSpec((2, _RB), lambda i: (0, i)),
        ],
        out_specs=pl.BlockSpec((_RB, F_HID), lambda i: (i, 0)),
        out_shape=jax.ShapeDtypeStruct((NP, F_HID), jnp.float32),
    )(xp, W1, degp)


def _tc2(h1p, degp, W2):
    return pl.pallas_call(
        _tc2_body,
        grid=(NP // _RB,),
        in_specs=[
            pl.BlockSpec((2, _RB, F_HID), lambda i: (0, i, 0)),
            pl.BlockSpec((2, _RB), lambda i: (0, i)),
            pl.BlockSpec((F_HID, N_CLASS), lambda i: (0, 0)),
        ],
        out_specs=pl.BlockSpec((_RB, N_CLASS), lambda i: (i, 0)),
        out_shape=jax.ShapeDtypeStruct((NP, N_CLASS), jnp.float32),
    )(h1p, degp, W2)


def _tc3(h2p, degp):
    return pl.pallas_call(
        _tc3_body,
        grid=(NP // _RB,),
        in_specs=[
            pl.BlockSpec((2, _RB, N_CLASS), lambda i: (0, i, 0)),
            pl.BlockSpec((2, _RB), lambda i: (0, i)),
        ],
        out_specs=pl.BlockSpec((_RB, N_CLASS), lambda i: (i, 0)),
        out_shape=jax.ShapeDtypeStruct((NP, N_CLASS), jnp.float32),
    )(h2p, degp)


@jax.jit
def kernel(x, edge_index, W1, W2):
    pad = jnp.full((EPAD - E,), PAD_ROW, jnp.int32)
    src2d = jnp.concatenate([edge_index[0], pad]).reshape(EPAD // CHUNK, CHUNK)
    dst2d = jnp.concatenate([edge_index[1], pad]).reshape(EPAD // CHUNK, CHUNK)
    xp = jnp.pad(x, ((0, NP - N), (0, 0)))
    z1 = jnp.zeros((NP,), jnp.float32)
    z64 = jnp.zeros((NP, F_HID), jnp.float32)
    z16 = jnp.zeros((NP, N_CLASS), jnp.float32)

    degp = _sc_degree(dst2d, z1)
    s1 = _tc1(xp, W1, degp)
    h1p = _sc_spmm64(s1, src2d, dst2d, z64)
    s2 = _tc2(h1p, degp, W2)
    h2p = _sc_spmm16(s2, src2d, dst2d, z16)
    out = _tc3(h2p, degp)
    return out[:N]


# speedup vs baseline: 1.1975x; 1.0334x over previous
import functools

import jax
import jax.numpy as jnp
from jax import lax
from jax.experimental import pallas as pl
from jax.experimental.pallas import tpu as pltpu
from jax.experimental.pallas import tpu_sc as plsc

N = 10000
E = 320000
F_IN = 128
F_HID = 64
N_CLASS = 16

NP = 10240
PAD_ROW = N
CHUNK = 128
EPAD = 327680
CPT = EPAD // (32 * CHUNK)
RPT = NP // 16

_MESH = plsc.VectorSubcoreMesh(core_axis_name="c", subcore_axis_name="s")
_SC_PARAMS = pltpu.CompilerParams(use_tc_tiling_on_sc=False)


@functools.partial(
    pl.kernel,
    out_type=jax.ShapeDtypeStruct((2, NP), jnp.float32),
    mesh=_MESH,
    compiler_params=_SC_PARAMS,
    scratch_types=[
        pltpu.VMEM((CPT, CHUNK), jnp.int32),
        pltpu.VMEM((CHUNK,), jnp.float32),
        pltpu.VMEM_SHARED((NP,), jnp.float32),
    ],
)
def _sc_degree(dst_hbm, z_hbm, out_hbm, dst_v, ones_v, acc):
    c = lax.axis_index("c")
    s = lax.axis_index("s")
    wid = c * 16 + s
    pltpu.sync_copy(z_hbm.at[pl.ds(s * RPT, RPT)], acc.at[pl.ds(s * RPT, RPT)])
    for i in range(CHUNK // 16):
        ones_v[pl.ds(i * 16, 16)] = jnp.full((16,), 1.0, jnp.float32)
    pltpu.sync_copy(dst_hbm.at[pl.ds(wid * CPT, CPT)], dst_v)
    plsc.subcore_barrier()

    def body(j, carry):
        pltpu.sync_copy(ones_v, acc.at[dst_v.at[j]], add=True)
        return carry

    lax.fori_loop(0, CPT, body, 0)
    plsc.subcore_barrier()
    pltpu.sync_copy(acc.at[pl.ds(s * RPT, RPT)], out_hbm.at[c, pl.ds(s * RPT, RPT)])


NBUF = 8
CPT0 = 128
CPT1 = 160 - CPT0


def _make_sc_spmm(width):
    @functools.partial(
        pl.kernel,
        out_type=jax.ShapeDtypeStruct((2, NP, width), jnp.float32),
        mesh=_MESH,
        compiler_params=_SC_PARAMS,
        scratch_types=[
            pltpu.VMEM((2, NBUF, CHUNK), jnp.int32),
            pltpu.VMEM((2, NBUF, CHUNK), jnp.int32),
            pltpu.VMEM((NBUF, CHUNK, width), jnp.float32),
            pltpu.VMEM_SHARED((NP, width), jnp.float32),
        ] + [pltpu.SemaphoreType.DMA] * (2 * NBUF + 2),
    )
    def spmm(s_hbm, src_hbm, dst_hbm, z_hbm, out_hbm,
             src_v, dst_v, rows_v, acc, *sems):
        gsems = sems[:NBUF]
        ssems = sems[NBUF:2 * NBUF]
        isems = sems[2 * NBUF:]
        c = lax.axis_index("c")
        s = lax.axis_index("s")
        pltpu.sync_copy(z_hbm.at[pl.ds(s * RPT, RPT)], acc.at[pl.ds(s * RPT, RPT)])
        plsc.subcore_barrier()

        def idx_load(slot, blk):
            pltpu.async_copy(src_hbm.at[pl.ds(blk * NBUF, NBUF)],
                             src_v.at[slot], isems[slot])
            pltpu.async_copy(dst_hbm.at[pl.ds(blk * NBUF, NBUF)],
                             dst_v.at[slot], isems[slot])

        def idx_wait(slot, blk):
            pltpu.make_async_copy(src_hbm.at[pl.ds(blk * NBUF, NBUF)],
                                  src_v.at[slot], isems[slot]).wait()
            pltpu.make_async_copy(dst_hbm.at[pl.ds(blk * NBUF, NBUF)],
                                  dst_v.at[slot], isems[slot]).wait()

        def run(base_blk, count):
            G = count // NBUF
            idx_load(0, base_blk)
            idx_load(1, base_blk + 1)
            idx_wait(0, base_blk)
            for b in range(NBUF):
                pltpu.async_copy(s_hbm.at[src_v.at[0, b]], rows_v.at[b],
                                 gsems[b])

            def group(g, slot):
                nslot = 1 - slot
                for b in range(NBUF):
                    pltpu.make_async_copy(
                        s_hbm.at[src_v.at[slot, b]], rows_v.at[b],
                        gsems[b]).wait()
                    pltpu.async_copy(
                        rows_v.at[b], acc.at[dst_v.at[slot, b]], ssems[b],
                        add=True)
                @pl.when(g + 1 < G)
                def _():
                    idx_wait(nslot, base_blk + g + 1)
                for b in range(NBUF):
                    pltpu.make_async_copy(
                        rows_v.at[b], acc.at[dst_v.at[slot, b]],
                        ssems[b]).wait()

                    @pl.when(g + 1 < G)
                    def _(b=b):
                        pltpu.async_copy(
                            s_hbm.at[src_v.at[nslot, b]], rows_v.at[b],
                            gsems[b])
                @pl.when(g + 2 < G)
                def _():
                    idx_load(slot, base_blk + g + 2)

            def body(h, carry):
                group(2 * h, 0)
                group(2 * h + 1, 1)
                return carry

            lax.fori_loop(0, G // 2, body, 0)

        @pl.when(c == 0)
        def _():
            run(s * (CPT0 // NBUF), CPT0)

        if CPT1:
            @pl.when(c == 1)
            def _():
                run((16 * CPT0 + s * CPT1) // NBUF, CPT1)
        plsc.subcore_barrier()
        pltpu.sync_copy(acc.at[pl.ds(s * RPT, RPT)],
                        out_hbm.at[c, pl.ds(s * RPT, RPT)])

    return spmm


_sc_spmm64 = _make_sc_spmm(F_HID)
_sc_spmm16 = _make_sc_spmm(N_CLASS)


_RB = 2048


def _dinv(d_ref):
    return lax.rsqrt(d_ref[0, :] + d_ref[1, :] + 1.0)


def _tc1_body(x_ref, w_ref, d_ref, o_ref):
    s = jnp.dot(x_ref[...], w_ref[...], preferred_element_type=jnp.float32)
    o_ref[...] = s * _dinv(d_ref)[:, None]


def _tc2_body(h_ref, d_ref, w_ref, o_ref):
    dinv = _dinv(d_ref)
    h1 = (h_ref[0] + h_ref[1]) * dinv[:, None]
    s = jnp.dot(h1, w_ref[...], preferred_element_type=jnp.float32)
    o_ref[...] = s * dinv[:, None]


def _tc3_body(h_ref, d_ref, o_ref):
    h2 = (h_ref[0] + h_ref[1]) * _dinv(d_ref)[:, None]
    m = jnp.max(h2, axis=1, keepdims=True)
    ex = jnp.exp(h2 - m)
    lse = jnp.log(jnp.sum(ex, axis=1, keepdims=True)) + m
    o_ref[...] = h2 - lse


def _tc1(xp, W1, degp):
    return pl.pallas_call(
        _tc1_body,
        grid=(NP // _RB,),
        in_specs=[
            pl.BlockSpec((_RB, F_IN), lambda i: (i, 0)),
            pl.BlockSpec((F_IN, F_HID), lambda i: (0, 0)),
            pl.BlockSpec((2, _RB), lambda i: (0, i)),
        ],
        out_specs=pl.BlockSpec((_RB, F_HID), lambda i: (i, 0)),
        out_shape=jax.ShapeDtypeStruct((NP, F_HID), jnp.float32),
    )(xp, W1, degp)


def _tc2(h1p, degp, W2):
    return pl.pallas_call(
        _tc2_body,
        grid=(NP // _RB,),
        in_specs=[
            pl.BlockSpec((2, _RB, F_HID), lambda i: (0, i, 0)),
            pl.BlockSpec((2, _RB), lambda i: (0, i)),
            pl.BlockSpec((F_HID, N_CLASS), lambda i: (0, 0)),
        ],
        out_specs=pl.BlockSpec((_RB, N_CLASS), lambda i: (i, 0)),
        out_shape=jax.ShapeDtypeStruct((NP, N_CLASS), jnp.float32),
    )(h1p, degp, W2)


def _tc3(h2p, degp):
    return pl.pallas_call(
        _tc3_body,
        grid=(NP // _RB,),
        in_specs=[
            pl.BlockSpec((2, _RB, N_CLASS), lambda i: (0, i, 0)),
            pl.BlockSpec((2, _RB), lambda i: (0, i)),
        ],
        out_specs=pl.BlockSpec((_RB, N_CLASS), lambda i: (i, 0)),
        out_shape=jax.ShapeDtypeStruct((NP, N_CLASS), jnp.float32),
    )(h2p, degp)


@jax.jit
def kernel(x, edge_index, W1, W2):
    pad = jnp.full((EPAD - E,), PAD_ROW, jnp.int32)
    src2d = jnp.concatenate([edge_index[0], pad]).reshape(EPAD // CHUNK, CHUNK)
    dst2d = jnp.concatenate([edge_index[1], pad]).reshape(EPAD // CHUNK, CHUNK)
    xp = jnp.pad(x, ((0, NP - N), (0, 0)))
    z1 = jnp.zeros((NP,), jnp.float32)
    z64 = jnp.zeros((NP, F_HID), jnp.float32)
    z16 = jnp.zeros((NP, N_CLASS), jnp.float32)

    degp = _sc_degree(dst2d, z1)
    s1 = _tc1(xp, W1, degp)
    h1p = _sc_spmm64(s1, src2d, dst2d, z64)
    s2 = _tc2(h1p, degp, W2)
    h2p = _sc_spmm16(s2, src2d, dst2d, z16)
    out = _tc3(h2p, degp)
    return out[:N]


# speedup vs baseline: 2.7631x; 2.3074x over previous
import functools

import jax
import jax.numpy as jnp
from jax import lax
from jax.experimental import pallas as pl
from jax.experimental.pallas import tpu as pltpu
from jax.experimental.pallas import tpu_sc as plsc

N = 10000
E = 320000
F_IN = 128
F_HID = 64
N_CLASS = 16

NP = 10240
PAD_ROW = N
CHUNK = 128
EPAD = 327680
CPT = EPAD // (32 * CHUNK)
RPT = NP // 16

_MESH = plsc.VectorSubcoreMesh(core_axis_name="c", subcore_axis_name="s")
_SC_PARAMS = pltpu.CompilerParams(use_tc_tiling_on_sc=False)


@functools.partial(
    pl.kernel,
    out_type=jax.ShapeDtypeStruct((2, NP), jnp.float32),
    mesh=_MESH,
    compiler_params=_SC_PARAMS,
    scratch_types=[
        pltpu.VMEM((CPT, CHUNK), jnp.int32),
        pltpu.VMEM((CHUNK,), jnp.float32),
        pltpu.VMEM_SHARED((NP,), jnp.float32),
    ],
)
def _sc_degree(dst_hbm, z_hbm, out_hbm, dst_v, ones_v, acc):
    c = lax.axis_index("c")
    s = lax.axis_index("s")
    wid = c * 16 + s
    pltpu.sync_copy(z_hbm.at[pl.ds(s * RPT, RPT)], acc.at[pl.ds(s * RPT, RPT)])
    for i in range(CHUNK // 16):
        ones_v[pl.ds(i * 16, 16)] = jnp.full((16,), 1.0, jnp.float32)
    pltpu.sync_copy(dst_hbm.at[pl.ds(wid * CPT, CPT)], dst_v)
    plsc.subcore_barrier()

    def body(j, carry):
        pltpu.sync_copy(ones_v, acc.at[dst_v.at[j]], add=True)
        return carry

    lax.fori_loop(0, CPT, body, 0)
    plsc.subcore_barrier()
    pltpu.sync_copy(acc.at[pl.ds(s * RPT, RPT)], out_hbm.at[c, pl.ds(s * RPT, RPT)])


NBUF = 8
CPT0 = 80
CPT1 = 160 - CPT0


def _make_sc_spmm(width):
    @functools.partial(
        pl.kernel,
        out_type=jax.ShapeDtypeStruct((2, NP, width), jnp.float32),
        mesh=_MESH,
        compiler_params=_SC_PARAMS,
        scratch_types=[
            pltpu.VMEM((2, NBUF, CHUNK), jnp.int32),
            pltpu.VMEM((2, NBUF, CHUNK), jnp.int32),
            pltpu.VMEM((NBUF, CHUNK, width), jnp.float32),
            pltpu.VMEM_SHARED((NP, width), jnp.float32),
        ] + [pltpu.SemaphoreType.DMA] * (2 * NBUF + 2),
    )
    def spmm(s_hbm, src_hbm, dst_hbm, z_hbm, out_hbm,
             src_v, dst_v, rows_v, acc, *sems):
        gsems = sems[:NBUF]
        ssems = sems[NBUF:2 * NBUF]
        isems = sems[2 * NBUF:]
        c = lax.axis_index("c")
        s = lax.axis_index("s")
        pltpu.sync_copy(z_hbm.at[pl.ds(s * RPT, RPT)], acc.at[pl.ds(s * RPT, RPT)])
        plsc.subcore_barrier()

        def idx_load(slot, blk):
            pltpu.async_copy(src_hbm.at[pl.ds(blk * NBUF, NBUF)],
                             src_v.at[slot], isems[slot])
            pltpu.async_copy(dst_hbm.at[pl.ds(blk * NBUF, NBUF)],
                             dst_v.at[slot], isems[slot])

        def idx_wait(slot, blk):
            pltpu.make_async_copy(src_hbm.at[pl.ds(blk * NBUF, NBUF)],
                                  src_v.at[slot], isems[slot]).wait()
            pltpu.make_async_copy(dst_hbm.at[pl.ds(blk * NBUF, NBUF)],
                                  dst_v.at[slot], isems[slot]).wait()

        def run(base_blk, count):
            G = count // NBUF
            idx_load(0, base_blk)
            idx_load(1, base_blk + 1)
            idx_wait(0, base_blk)
            for b in range(NBUF):
                pltpu.async_copy(s_hbm.at[src_v.at[0, b]], rows_v.at[b],
                                 gsems[b])

            def group(g, slot):
                nslot = 1 - slot
                for b in range(NBUF):
                    pltpu.make_async_copy(
                        s_hbm.at[src_v.at[slot, b]], rows_v.at[b],
                        gsems[b]).wait()
                    pltpu.async_copy(
                        rows_v.at[b], acc.at[dst_v.at[slot, b]], ssems[b],
                        add=True)
                @pl.when(g + 1 < G)
                def _():
                    idx_wait(nslot, base_blk + g + 1)
                for b in range(NBUF):
                    pltpu.make_async_copy(
                        rows_v.at[b], acc.at[dst_v.at[slot, b]],
                        ssems[b]).wait()

                    @pl.when(g + 1 < G)
                    def _(b=b):
                        pltpu.async_copy(
                            s_hbm.at[src_v.at[nslot, b]], rows_v.at[b],
                            gsems[b])
                @pl.when(g + 2 < G)
                def _():
                    idx_load(slot, base_blk + g + 2)

            def body(h, carry):
                group(2 * h, 0)
                group(2 * h + 1, 1)
                return carry

            lax.fori_loop(0, G // 2, body, 0)

        @pl.when(c == 0)
        def _():
            run(s * (CPT0 // NBUF), CPT0)

        if CPT1:
            @pl.when(c == 1)
            def _():
                run((16 * CPT0 + s * CPT1) // NBUF, CPT1)
        plsc.subcore_barrier()
        pltpu.sync_copy(acc.at[pl.ds(s * RPT, RPT)],
                        out_hbm.at[c, pl.ds(s * RPT, RPT)])

    return spmm


_sc_spmm64 = _make_sc_spmm(F_HID)
_sc_spmm16 = _make_sc_spmm(N_CLASS)


_RB = 2048


def _dinv(d_ref):
    return lax.rsqrt(d_ref[0, :] + d_ref[1, :] + 1.0)


def _tc1_body(x_ref, w_ref, d_ref, o_ref):
    s = jnp.dot(x_ref[...], w_ref[...], preferred_element_type=jnp.float32)
    o_ref[...] = s * _dinv(d_ref)[:, None]


def _tc2_body(h_ref, d_ref, w_ref, o_ref):
    dinv = _dinv(d_ref)
    h1 = (h_ref[0] + h_ref[1]) * dinv[:, None]
    s = jnp.dot(h1, w_ref[...], preferred_element_type=jnp.float32)
    o_ref[...] = s * dinv[:, None]


def _tc3_body(h_ref, d_ref, o_ref):
    h2 = (h_ref[0] + h_ref[1]) * _dinv(d_ref)[:, None]
    m = jnp.max(h2, axis=1, keepdims=True)
    ex = jnp.exp(h2 - m)
    lse = jnp.log(jnp.sum(ex, axis=1, keepdims=True)) + m
    o_ref[...] = h2 - lse


def _tc1(xp, W1, degp):
    return pl.pallas_call(
        _tc1_body,
        grid=(NP // _RB,),
        in_specs=[
            pl.BlockSpec((_RB, F_IN), lambda i: (i, 0)),
            pl.BlockSpec((F_IN, F_HID), lambda i: (0, 0)),
            pl.BlockSpec((2, _RB), lambda i: (0, i)),
        ],
        out_specs=pl.BlockSpec((_RB, F_HID), lambda i: (i, 0)),
        out_shape=jax.ShapeDtypeStruct((NP, F_HID), jnp.float32),
    )(xp, W1, degp)


def _tc2(h1p, degp, W2):
    return pl.pallas_call(
        _tc2_body,
        grid=(NP // _RB,),
        in_specs=[
            pl.BlockSpec((2, _RB, F_HID), lambda i: (0, i, 0)),
            pl.BlockSpec((2, _RB), lambda i: (0, i)),
            pl.BlockSpec((F_HID, N_CLASS), lambda i: (0, 0)),
        ],
        out_specs=pl.BlockSpec((_RB, N_CLASS), lambda i: (i, 0)),
        out_shape=jax.ShapeDtypeStruct((NP, N_CLASS), jnp.float32),
    )(h1p, degp, W2)


def _tc3(h2p, degp):
    return pl.pallas_call(
        _tc3_body,
        grid=(NP // _RB,),
        in_specs=[
            pl.BlockSpec((2, _RB, N_CLASS), lambda i: (0, i, 0)),
            pl.BlockSpec((2, _RB), lambda i: (0, i)),
        ],
        out_specs=pl.BlockSpec((_RB, N_CLASS), lambda i: (i, 0)),
        out_shape=jax.ShapeDtypeStruct((NP, N_CLASS), jnp.float32),
    )(h2p, degp)


@jax.jit
def kernel(x, edge_index, W1, W2):
    pad = PAD_ROW + jnp.arange(EPAD - E, dtype=jnp.int32) % (NP - N)
    src2d = jnp.concatenate([edge_index[0], pad]).reshape(EPAD // CHUNK, CHUNK)
    dst2d = jnp.concatenate([edge_index[1], pad]).reshape(EPAD // CHUNK, CHUNK)
    xp = jnp.pad(x, ((0, NP - N), (0, 0)))
    z1 = jnp.zeros((NP,), jnp.float32)
    z64 = jnp.zeros((NP, F_HID), jnp.float32)
    z16 = jnp.zeros((NP, N_CLASS), jnp.float32)

    degp = _sc_degree(dst2d, z1)
    s1 = _tc1(xp, W1, degp)
    h1p = _sc_spmm64(s1, src2d, dst2d, z64)
    s2 = _tc2(h1p, degp, W2)
    h2p = _sc_spmm16(s2, src2d, dst2d, z16)
    out = _tc3(h2p, degp)
    return out[:N]


# speedup vs baseline: 2.8373x; 1.0268x over previous
import functools

import jax
import jax.numpy as jnp
from jax import lax
from jax.experimental import pallas as pl
from jax.experimental.pallas import tpu as pltpu
from jax.experimental.pallas import tpu_sc as plsc

N = 10000
E = 320000
F_IN = 128
F_HID = 64
N_CLASS = 16

NP = 10240
PAD_ROW = N
CHUNK = 128
EPAD = 327680
CPT = EPAD // (32 * CHUNK)
RPT = NP // 16

_MESH = plsc.VectorSubcoreMesh(core_axis_name="c", subcore_axis_name="s")
_SC_PARAMS = pltpu.CompilerParams(use_tc_tiling_on_sc=False)


DCPT = 78
DTAIL = E // CHUNK - 32 * DCPT


@functools.partial(
    pl.kernel,
    out_type=jax.ShapeDtypeStruct((2, NP), jnp.float32),
    mesh=_MESH,
    compiler_params=_SC_PARAMS,
    scratch_types=[
        pltpu.VMEM((DCPT + 1, CHUNK), jnp.int32),
        pltpu.VMEM((CHUNK,), jnp.float32),
        pltpu.VMEM_SHARED((NP,), jnp.float32),
        pltpu.SemaphoreType.DMA,
    ],
)
def _sc_degree(ei_hbm, z_hbm, out_hbm, dst_v, ones_v, acc, sem):
    c = lax.axis_index("c")
    s = lax.axis_index("s")
    wid = c * 16 + s
    pltpu.sync_copy(z_hbm.at[pl.ds(s * RPT, RPT)], acc.at[pl.ds(s * RPT, RPT)])
    for i in range(CHUNK // 16):
        ones_v[pl.ds(i * 16, 16)] = jnp.full((16,), 1.0, jnp.float32)
    pltpu.sync_copy(ei_hbm.at[1, pl.ds(wid * DCPT, DCPT)],
                    dst_v.at[pl.ds(0, DCPT)])

    @pl.when(wid < DTAIL)
    def _():
        pltpu.sync_copy(ei_hbm.at[1, pl.ds(32 * DCPT + wid, 1)],
                        dst_v.at[pl.ds(DCPT, 1)])
    plsc.subcore_barrier()

    def fire(j, carry):
        pltpu.async_copy(ones_v, acc.at[dst_v.at[j]], sem, add=True)
        return carry

    def drain(j, carry):
        pltpu.make_async_copy(ones_v, acc.at[dst_v.at[j]], sem).wait()
        return carry

    lax.fori_loop(0, DCPT, fire, 0)

    @pl.when(wid < DTAIL)
    def _():
        pltpu.async_copy(ones_v, acc.at[dst_v.at[DCPT]], sem, add=True)
    lax.fori_loop(0, DCPT, drain, 0)

    @pl.when(wid < DTAIL)
    def _():
        pltpu.make_async_copy(ones_v, acc.at[dst_v.at[DCPT]], sem).wait()
    plsc.subcore_barrier()
    pltpu.sync_copy(acc.at[pl.ds(s * RPT, RPT)], out_hbm.at[c, pl.ds(s * RPT, RPT)])


NBUF = 10
CPT0 = 80
CPT1 = 160 - CPT0


def _make_sc_spmm(width):
    @functools.partial(
        pl.kernel,
        out_type=jax.ShapeDtypeStruct((2, NP, width), jnp.float32),
        mesh=_MESH,
        compiler_params=_SC_PARAMS,
        scratch_types=[
            pltpu.VMEM((2, NBUF, CHUNK), jnp.int32),
            pltpu.VMEM((2, NBUF, CHUNK), jnp.int32),
            pltpu.VMEM((NBUF, CHUNK, width), jnp.float32),
            pltpu.VMEM_SHARED((NP, width), jnp.float32),
        ] + [pltpu.SemaphoreType.DMA] * (2 * NBUF + 2),
    )
    def spmm(s_hbm, src_hbm, dst_hbm, z_hbm, out_hbm,
             src_v, dst_v, rows_v, acc, *sems):
        gsems = sems[:NBUF]
        ssems = sems[NBUF:2 * NBUF]
        isems = sems[2 * NBUF:]
        c = lax.axis_index("c")
        s = lax.axis_index("s")
        pltpu.sync_copy(z_hbm.at[pl.ds(s * RPT, RPT)], acc.at[pl.ds(s * RPT, RPT)])
        plsc.subcore_barrier()

        def idx_load(slot, blk):
            pltpu.async_copy(src_hbm.at[pl.ds(blk * NBUF, NBUF)],
                             src_v.at[slot], isems[slot])
            pltpu.async_copy(dst_hbm.at[pl.ds(blk * NBUF, NBUF)],
                             dst_v.at[slot], isems[slot])

        def idx_wait(slot, blk):
            pltpu.make_async_copy(src_hbm.at[pl.ds(blk * NBUF, NBUF)],
                                  src_v.at[slot], isems[slot]).wait()
            pltpu.make_async_copy(dst_hbm.at[pl.ds(blk * NBUF, NBUF)],
                                  dst_v.at[slot], isems[slot]).wait()

        def run(base_blk, count):
            G = count // NBUF
            idx_load(0, base_blk)
            idx_load(1, base_blk + 1)
            idx_wait(0, base_blk)
            for b in range(NBUF):
                pltpu.async_copy(s_hbm.at[src_v.at[0, b]], rows_v.at[b],
                                 gsems[b])

            def group(g, slot):
                nslot = 1 - slot
                for b in range(NBUF):
                    pltpu.make_async_copy(
                        s_hbm.at[src_v.at[slot, b]], rows_v.at[b],
                        gsems[b]).wait()
                    pltpu.async_copy(
                        rows_v.at[b], acc.at[dst_v.at[slot, b]], ssems[b],
                        add=True)
                @pl.when(g + 1 < G)
                def _():
                    idx_wait(nslot, base_blk + g + 1)
                for b in range(NBUF):
                    pltpu.make_async_copy(
                        rows_v.at[b], acc.at[dst_v.at[slot, b]],
                        ssems[b]).wait()

                    @pl.when(g + 1 < G)
                    def _(b=b):
                        pltpu.async_copy(
                            s_hbm.at[src_v.at[nslot, b]], rows_v.at[b],
                            gsems[b])
                @pl.when(g + 2 < G)
                def _():
                    idx_load(slot, base_blk + g + 2)

            def body(h, carry):
                group(2 * h, 0)
                group(2 * h + 1, 1)
                return carry

            lax.fori_loop(0, G // 2, body, 0)

        @pl.when(c == 0)
        def _():
            run(s * (CPT0 // NBUF), CPT0)

        if CPT1:
            @pl.when(c == 1)
            def _():
                run((16 * CPT0 + s * CPT1) // NBUF, CPT1)
        plsc.subcore_barrier()
        pltpu.sync_copy(acc.at[pl.ds(s * RPT, RPT)],
                        out_hbm.at[c, pl.ds(s * RPT, RPT)])

    return spmm


_sc_spmm64 = _make_sc_spmm(F_HID)
_sc_spmm16 = _make_sc_spmm(N_CLASS)


_RB = 2048


def _dinv(d_ref):
    return lax.rsqrt(d_ref[0, :] + d_ref[1, :] + 1.0)


def _tc1_body(x_ref, w_ref, d_ref, o_ref):
    s = jnp.dot(x_ref[...], w_ref[...], preferred_element_type=jnp.float32)
    o_ref[...] = s * _dinv(d_ref)[:, None]


def _tc2_body(h_ref, d_ref, w_ref, o_ref):
    dinv = _dinv(d_ref)
    h1 = (h_ref[0] + h_ref[1]) * dinv[:, None]
    s = jnp.dot(h1, w_ref[...], preferred_element_type=jnp.float32)
    o_ref[...] = s * dinv[:, None]


def _tc3_body(h_ref, d_ref, o_ref):
    h2 = (h_ref[0] + h_ref[1]) * _dinv(d_ref)[:, None]
    m = jnp.max(h2, axis=1, keepdims=True)
    ex = jnp.exp(h2 - m)
    lse = jnp.log(jnp.sum(ex, axis=1, keepdims=True)) + m
    o_ref[...] = h2 - lse


def _tc1(xp, W1, degp):
    return pl.pallas_call(
        _tc1_body,
        grid=(NP // _RB,),
        in_specs=[
            pl.BlockSpec((_RB, F_IN), lambda i: (i, 0)),
            pl.BlockSpec((F_IN, F_HID), lambda i: (0, 0)),
            pl.BlockSpec((2, _RB), lambda i: (0, i)),
        ],
        out_specs=pl.BlockSpec((_RB, F_HID), lambda i: (i, 0)),
        out_shape=jax.ShapeDtypeStruct((NP, F_HID), jnp.float32),
    )(xp, W1, degp)


def _tc2(h1p, degp, W2):
    return pl.pallas_call(
        _tc2_body,
        grid=(NP // _RB,),
        in_specs=[
            pl.BlockSpec((2, _RB, F_HID), lambda i: (0, i, 0)),
            pl.BlockSpec((2, _RB), lambda i: (0, i)),
            pl.BlockSpec((F_HID, N_CLASS), lambda i: (0, 0)),
        ],
        out_specs=pl.BlockSpec((_RB, N_CLASS), lambda i: (i, 0)),
        out_shape=jax.ShapeDtypeStruct((NP, N_CLASS), jnp.float32),
    )(h1p, degp, W2)


def _tc3(h2p, degp):
    return pl.pallas_call(
        _tc3_body,
        grid=(NP // _RB,),
        in_specs=[
            pl.BlockSpec((2, _RB, N_CLASS), lambda i: (0, i, 0)),
            pl.BlockSpec((2, _RB), lambda i: (0, i)),
        ],
        out_specs=pl.BlockSpec((_RB, N_CLASS), lambda i: (i, 0)),
        out_shape=jax.ShapeDtypeStruct((NP, N_CLASS), jnp.float32),
    )(h2p, degp)


@jax.jit
def kernel(x, edge_index, W1, W2):
    pad = PAD_ROW + jnp.arange(EPAD - E, dtype=jnp.int32) % (NP - N)
    src2d = jnp.concatenate([edge_index[0], pad]).reshape(EPAD // CHUNK, CHUNK)
    dst2d = jnp.concatenate([edge_index[1], pad]).reshape(EPAD // CHUNK, CHUNK)
    xp = jnp.pad(x, ((0, NP - N), (0, 0)))
    z1 = jnp.zeros((NP,), jnp.float32)
    z64 = jnp.zeros((NP, F_HID), jnp.float32)
    z16 = jnp.zeros((NP, N_CLASS), jnp.float32)

    ei3 = edge_index.reshape(2, E // CHUNK, CHUNK)
    degp = _sc_degree(ei3, z1)
    s1 = _tc1(xp, W1, degp)
    h1p = _sc_spmm64(s1, src2d, dst2d, z64)
    s2 = _tc2(h1p, degp, W2)
    h2p = _sc_spmm16(s2, src2d, dst2d, z16)
    out = _tc3(h2p, degp)
    return out[:N]
